# baseline (speedup 1.0000x reference)
import math

import jax
import jax.numpy as jnp
from jax.experimental import pallas as pl
from jax.experimental.pallas import tpu as pltpu

DIM = 1024
CAPACITY = 0.25


def _mask_kernel(x_ref, wr_ref, mask_ref, *, L, K, C):
    xb = x_ref[0].astype(jnp.bfloat16).astype(jnp.float32)
    wr = wr_ref[:].astype(jnp.bfloat16).astype(jnp.float32)
    D = xb.shape[1]
    r_col = jax.lax.dot_general(
        xb[:, 0:256], wr[:, 0:256], (((1,), (1,)), ((), ())),
        preferred_element_type=jnp.float32)
    for c in range(256, D, 256):
        r_col = r_col + jax.lax.dot_general(
            xb[:, c:c + 256], wr[:, c:c + 256], (((1,), (1,)), ((), ())),
            preferred_element_type=jnp.float32)
    row_chunks = []
    for c0 in range(0, L, C):
        rows = jax.lax.broadcasted_iota(jnp.int32, (L, C), 0)
        cols = jax.lax.broadcasted_iota(jnp.int32, (L, C), 1) + c0
        ident = (rows == cols).astype(jnp.float32)
        row_chunks.append(jax.lax.dot_general(
            r_col, ident, (((0,), (0,)), ((), ())),
            precision=jax.lax.Precision.HIGHEST,
            preferred_element_type=jnp.float32))
    r_row = jnp.concatenate(row_chunks, axis=1)
    idx_row = jax.lax.broadcasted_iota(jnp.int32, (1, L), 1)
    for c0 in range(0, L, C):
        r_c = r_col[c0:c0 + C, :]
        gt = (r_row > r_c).astype(jnp.float32)
        idx_c = jax.lax.broadcasted_iota(jnp.int32, (C, 1), 0) + c0
        eqb = ((r_row == r_c) & (idx_row < idx_c)).astype(jnp.float32)
        rank = jnp.sum(gt + eqb, axis=1, keepdims=True)
        mask_ref[0, pl.ds(c0, C), :] = (rank < K).astype(jnp.float32)


def _block_kernel(xt_ref, xm_ref, mask_ref, g1_ref, b1_ref, g2_ref, b2_ref,
                  wfc_ref, bfc_ref, wproj_ref, bproj_ref, out_ref, *, T):
    xt = xt_ref[0]
    xm = xm_ref[0]
    mu = jnp.mean(xm, axis=-1, keepdims=True)
    var = jnp.mean((xm - mu) ** 2, axis=-1, keepdims=True)
    xn_m = (xm - mu) * jax.lax.rsqrt(var + 1e-5) * g1_ref[:] + b1_ref[:]
    F = 256
    prow = jax.lax.broadcasted_iota(jnp.int32, (F, F), 0)
    pcol = jax.lax.broadcasted_iota(jnp.int32, (F, F), 1)
    pmat = (prow + pcol == F - 1).astype(jnp.float32)
    flips = [jax.lax.dot_general(
        pmat, xn_m[c:c + F, :], (((1,), (0,)), ((), ())),
        preferred_element_type=jnp.float32) for c in range(0, T, F)]
    xn_f = jnp.concatenate(flips[::-1], axis=0)
    m = mask_ref[0]
    x1 = xt + m * xn_f
    mu2 = jnp.mean(x1, axis=-1, keepdims=True)
    var2 = jnp.mean((x1 - mu2) ** 2, axis=-1, keepdims=True)
    xn2 = (x1 - mu2) * jax.lax.rsqrt(var2 + 1e-5) * g2_ref[:] + b2_ref[:]
    a = math.sqrt(2.0 / math.pi)
    b_ = a * 0.044715
    h = jax.lax.dot_general(
        xn2, wfc_ref[:], (((1,), (1,)), ((), ())),
        preferred_element_type=jnp.float32) + bfc_ref[:]
    t = jnp.tanh(h * (a + b_ * (h * h)))
    hh = 0.5 * h
    g = hh + hh * t
    y = jax.lax.dot_general(
        g, wproj_ref[:], (((1,), (1,)), ((), ())),
        preferred_element_type=jnp.float32) + bproj_ref[:]
    out_ref[0] = x1 + y


def kernel(x, Wr, br, g1, b1, Wq, Wk, Wv, Wfc, bfc, Wproj, bproj, g2, b2):
    b, l, d = x.shape
    k = math.ceil(l * CAPACITY)
    inner = Wfc.shape[0]

    mask = pl.pallas_call(
        lambda x_ref, wr_ref, mask_ref: _mask_kernel(
            x_ref, wr_ref, mask_ref, L=l, K=k, C=256),
        grid=(b,),
        in_specs=[
            pl.BlockSpec((1, l, d), lambda i: (i, 0, 0)),
            pl.BlockSpec((1, d), lambda i: (0, 0)),
        ],
        out_specs=pl.BlockSpec((1, l, 1), lambda i: (i, 0, 0)),
        out_shape=jax.ShapeDtypeStruct((b, l, 1), jnp.float32),
    )(x, Wr)

    T = 512
    nt = l // T
    out = pl.pallas_call(
        lambda *refs: _block_kernel(*refs, T=T),
        grid=(b, nt),
        in_specs=[
            pl.BlockSpec((1, T, d), lambda i, j: (i, j, 0)),
            pl.BlockSpec((1, T, d), lambda i, j: (i, nt - 1 - j, 0)),
            pl.BlockSpec((1, T, 1), lambda i, j: (i, j, 0)),
            pl.BlockSpec((1, d), lambda i, j: (0, 0)),
            pl.BlockSpec((1, d), lambda i, j: (0, 0)),
            pl.BlockSpec((1, d), lambda i, j: (0, 0)),
            pl.BlockSpec((1, d), lambda i, j: (0, 0)),
            pl.BlockSpec((inner, d), lambda i, j: (0, 0)),
            pl.BlockSpec((1, inner), lambda i, j: (0, 0)),
            pl.BlockSpec((d, inner), lambda i, j: (0, 0)),
            pl.BlockSpec((1, d), lambda i, j: (0, 0)),
        ],
        out_specs=pl.BlockSpec((1, T, d), lambda i, j: (i, j, 0)),
        out_shape=jax.ShapeDtypeStruct((b, l, d), jnp.float32),
        compiler_params=pltpu.CompilerParams(
            vmem_limit_bytes=100 * 1024 * 1024,
            dimension_semantics=("parallel", "arbitrary")),
    )(x, x, mask, g1.reshape(1, d), b1.reshape(1, d),
      g2.reshape(1, d), b2.reshape(1, d),
      Wfc, bfc.reshape(1, inner), Wproj, bproj.reshape(1, d))
    return out


# speedup vs baseline: 1.0078x; 1.0078x over previous
import math

import jax
import jax.numpy as jnp
from jax.experimental import pallas as pl
from jax.experimental.pallas import tpu as pltpu

DIM = 1024
CAPACITY = 0.25


def _mask_kernel(x_ref, wr_ref, mask_ref, *, L, K, C):
    xb = x_ref[0].astype(jnp.bfloat16).astype(jnp.float32)
    wr = wr_ref[:].astype(jnp.bfloat16).astype(jnp.float32)
    D = xb.shape[1]
    r_col = jax.lax.dot_general(
        xb[:, 0:256], wr[:, 0:256], (((1,), (1,)), ((), ())),
        preferred_element_type=jnp.float32)
    for c in range(256, D, 256):
        r_col = r_col + jax.lax.dot_general(
            xb[:, c:c + 256], wr[:, c:c + 256], (((1,), (1,)), ((), ())),
            preferred_element_type=jnp.float32)
    row_chunks = []
    for c0 in range(0, L, C):
        rows = jax.lax.broadcasted_iota(jnp.int32, (L, C), 0)
        cols = jax.lax.broadcasted_iota(jnp.int32, (L, C), 1) + c0
        ident = (rows == cols).astype(jnp.float32)
        row_chunks.append(jax.lax.dot_general(
            r_col, ident, (((0,), (0,)), ((), ())),
            precision=jax.lax.Precision.HIGHEST,
            preferred_element_type=jnp.float32))
    r_row = jnp.concatenate(row_chunks, axis=1)
    idx_row = jax.lax.broadcasted_iota(jnp.int32, (1, L), 1)
    for c0 in range(0, L, C):
        r_c = r_col[c0:c0 + C, :]
        gt = (r_row > r_c).astype(jnp.float32)
        idx_c = jax.lax.broadcasted_iota(jnp.int32, (C, 1), 0) + c0
        eqb = ((r_row == r_c) & (idx_row < idx_c)).astype(jnp.float32)
        rank = jnp.sum(gt + eqb, axis=1, keepdims=True)
        mask_ref[0, pl.ds(c0, C), :] = (rank < K).astype(jnp.float32)


def _block_kernel(xt_ref, xm_ref, mask_ref, g1_ref, b1_ref, g2_ref, b2_ref,
                  wfc_ref, bfc_ref, wproj_ref, bproj_ref, out_ref, *, T):
    xt = xt_ref[0]
    xm = xm_ref[0]
    mu = jnp.mean(xm, axis=-1, keepdims=True)
    var = jnp.mean((xm - mu) ** 2, axis=-1, keepdims=True)
    xn_m = (xm - mu) * jax.lax.rsqrt(var + 1e-5) * g1_ref[:] + b1_ref[:]
    F = 256
    prow = jax.lax.broadcasted_iota(jnp.int32, (F, F), 0)
    pcol = jax.lax.broadcasted_iota(jnp.int32, (F, F), 1)
    pmat = (prow + pcol == F - 1).astype(jnp.float32)
    flips = [jax.lax.dot_general(
        pmat, xn_m[c:c + F, :], (((1,), (0,)), ((), ())),
        preferred_element_type=jnp.float32) for c in range(0, T, F)]
    xn_f = jnp.concatenate(flips[::-1], axis=0)
    m = mask_ref[0]
    x1 = xt + m * xn_f
    mu2 = jnp.mean(x1, axis=-1, keepdims=True)
    var2 = jnp.mean((x1 - mu2) ** 2, axis=-1, keepdims=True)
    xn2 = (x1 - mu2) * jax.lax.rsqrt(var2 + 1e-5) * g2_ref[:] + b2_ref[:]
    a = math.sqrt(2.0 / math.pi)
    b_ = a * 0.044715
    h = jax.lax.dot_general(
        xn2, wfc_ref[:], (((1,), (1,)), ((), ())),
        preferred_element_type=jnp.float32) + bfc_ref[:]
    t = jnp.tanh(h * (a + b_ * (h * h)))
    hh = 0.5 * h
    g = hh + hh * t
    y = jax.lax.dot_general(
        g, wproj_ref[:], (((1,), (1,)), ((), ())),
        preferred_element_type=jnp.float32) + bproj_ref[:]
    out_ref[0] = x1 + y


def kernel(x, Wr, br, g1, b1, Wq, Wk, Wv, Wfc, bfc, Wproj, bproj, g2, b2):
    b, l, d = x.shape
    k = math.ceil(l * CAPACITY)
    inner = Wfc.shape[0]

    mask = pl.pallas_call(
        lambda x_ref, wr_ref, mask_ref: _mask_kernel(
            x_ref, wr_ref, mask_ref, L=l, K=k, C=256),
        grid=(b,),
        in_specs=[
            pl.BlockSpec((1, l, d), lambda i: (i, 0, 0)),
            pl.BlockSpec((1, d), lambda i: (0, 0)),
        ],
        out_specs=pl.BlockSpec((1, l, 1), lambda i: (i, 0, 0)),
        out_shape=jax.ShapeDtypeStruct((b, l, 1), jnp.float32),
    )(x, Wr)

    T = 512
    nt = l // T
    out = pl.pallas_call(
        lambda *refs: _block_kernel(*refs, T=T),
        grid=(b, nt),
        in_specs=[
            pl.BlockSpec((1, T, d), lambda i, j: (i, j, 0)),
            pl.BlockSpec((1, T, d), lambda i, j: (i, nt - 1 - j, 0)),
            pl.BlockSpec((1, T, 1), lambda i, j: (i, j, 0)),
            pl.BlockSpec((1, d), lambda i, j: (0, 0)),
            pl.BlockSpec((1, d), lambda i, j: (0, 0)),
            pl.BlockSpec((1, d), lambda i, j: (0, 0)),
            pl.BlockSpec((1, d), lambda i, j: (0, 0)),
            pl.BlockSpec((inner, d), lambda i, j: (0, 0)),
            pl.BlockSpec((1, inner), lambda i, j: (0, 0)),
            pl.BlockSpec((d, inner), lambda i, j: (0, 0)),
            pl.BlockSpec((1, d), lambda i, j: (0, 0)),
        ],
        out_specs=pl.BlockSpec((1, T, d), lambda i, j: (i, j, 0)),
        out_shape=jax.ShapeDtypeStruct((b, l, d), jnp.float32),
        compiler_params=pltpu.CompilerParams(
            vmem_limit_bytes=100 * 1024 * 1024),
    )(x, x, mask, g1.reshape(1, d), b1.reshape(1, d),
      g2.reshape(1, d), b2.reshape(1, d),
      Wfc, bfc.reshape(1, inner), Wproj, bproj.reshape(1, d))
    return out
